# baseline (speedup 1.0000x reference)
import jax
import jax.numpy as jnp
from jax import lax
from jax.experimental import pallas as pl
from jax.experimental.pallas import tpu as pltpu
from jax.experimental.pallas import tpu_sc as plsc

_N = 10000
_E = 320000
_D = 128
_NC = 2
_NS = 16
_NW = _NC * _NS
_CH = 128
_NCHUNK = 80
_EPAD = _NW * _NCHUNK * _CH
_IB = 8
_NBLK = _NCHUNK // _IB
_K0 = 160
_K1 = 2 * _NCHUNK - _K0
_NP = 10240
_RPT = _NP // _NS
_ZC = 128
_L = 16

_mesh = plsc.VectorSubcoreMesh(core_axis_name="c", subcore_axis_name="s")


def _fill_rows(rows_v, value):
    vv = jnp.full((_L,), value, jnp.float32)

    def frow(i, _):
        def fcol(j, _):
            rows_v[i, pl.ds(j * _L, _L)] = vv
            return 0
        return lax.fori_loop(0, _D // _L, fcol, 0)
    lax.fori_loop(0, _ZC, frow, 0)


def _make_sc_agg(count_only):
    def body(table, src, dst, out_p, src_v, dst_v, rows0, rows1, acc,
             sem0, sem1):
        c = lax.axis_index("c")
        s = lax.axis_index("s")
        wid = s * _NC + c
        base = s * _RPT

        _fill_rows(rows0, 0.0)
        for k in range(_RPT // _ZC):
            pltpu.sync_copy(rows0, acc.at[pl.ds(base + k * _ZC, _ZC)])
        if count_only:
            _fill_rows(rows0, 1.0)
        plsc.subcore_barrier()

        rows = (rows0, rows1)
        sems = (sem0, sem1)

        if count_only:
            def block(b, _):
                pltpu.sync_copy(dst.at[pl.ds(wid * _NCHUNK + b * _IB, _IB)],
                                dst_v.at[0])

                def chunk(j, _):
                    pltpu.sync_copy(rows0, acc.at[dst_v.at[0, j]], add=True)
                    return 0
                lax.fori_loop(0, _IB, chunk, 0)
                return 0
            lax.fori_loop(0, _NBLK, block, 0)
        else:
            nblk = jnp.where(c == 0, _K0 // _IB, _K1 // _IB)
            start = jnp.where(c == 0, s * _K0, _NS * _K0 + s * _K1)

            def stage(b):
                ibase = start + b * _IB
                pltpu.sync_copy(src.at[pl.ds(ibase, _IB)], src_v.at[b % 2])
                pltpu.sync_copy(dst.at[pl.ds(ibase, _IB)], dst_v.at[b % 2])

            def issue(b, j, par):
                return pltpu.async_copy(table.at[src_v.at[b % 2, j]],
                                        rows[par], sems[par])

            @pl.when(nblk > 0)
            def _():
                stage(0)
                issue(0, 0, 0)

            def block(b, _):
                @pl.when(b + 1 < nblk)
                def _():
                    stage(b + 1)
                for j in range(_IB):
                    par = j % 2
                    if j + 1 < _IB:
                        issue(b, j + 1, (j + 1) % 2)
                    else:
                        @pl.when(b + 1 < nblk)
                        def _():
                            issue(b + 1, 0, (j + 1) % 2)
                    pltpu.make_async_copy(table.at[src_v.at[b % 2, j]],
                                          rows[par], sems[par]).wait()
                    pltpu.sync_copy(rows[par],
                                    acc.at[dst_v.at[b % 2, j]], add=True)
                return 0
            lax.fori_loop(0, nblk, block, 0)
        plsc.subcore_barrier()

        pltpu.sync_copy(acc.at[pl.ds(base, _RPT)],
                        out_p.at[c, pl.ds(base, _RPT)])

    return pl.kernel(
        body,
        out_type=[jax.ShapeDtypeStruct((_NC, _NP, _D), jnp.float32)],
        mesh=_mesh,
        scratch_types=[
            pltpu.VMEM((2, _IB, _CH), jnp.int32),
            pltpu.VMEM((2, _IB, _CH), jnp.int32),
            pltpu.VMEM((_ZC, _D), jnp.float32),
            pltpu.VMEM((_ZC, _D), jnp.float32),
            pltpu.VMEM_SHARED((_NP, _D), jnp.float32),
            pltpu.SemaphoreType.DMA,
            pltpu.SemaphoreType.DMA,
        ],
    )


_sc_agg = _make_sc_agg(False)
_sc_count = _make_sc_agg(True)

_BR = 1000
_dn = (((1,), (1,)), ((), ()))


def _tc1_body(p_ref, c_ref, x_ref, wl_ref, bl_ref, wr_ref, h_ref, inv_ref):
    ps = p_ref[0] + p_ref[1]
    cn = c_ref[0, :, 0:1] + c_ref[1, :, 0:1]
    inv = 1.0 / jnp.maximum(cn, 1.0)
    agg = ps * inv
    y = lax.dot_general(agg, wl_ref[...], _dn, preferred_element_type=jnp.float32)
    y = y + bl_ref[...] + lax.dot_general(x_ref[...], wr_ref[...], _dn,
                                          preferred_element_type=jnp.float32)
    h_ref[...] = jnp.maximum(y, 0.0)
    inv_ref[...] = inv


def _tc2_body(p_ref, inv_ref, x_ref, wl_ref, bl_ref, wr_ref, o_ref):
    agg = (p_ref[0] + p_ref[1]) * inv_ref[...]
    y = lax.dot_general(agg, wl_ref[...], _dn, preferred_element_type=jnp.float32)
    y = y + bl_ref[...] + lax.dot_general(x_ref[...], wr_ref[...], _dn,
                                          preferred_element_type=jnp.float32)
    o_ref[...] = y


_w_spec = pl.BlockSpec((_D, _D), lambda i: (0, 0))
_b_spec = pl.BlockSpec((1, _D), lambda i: (0, 0))
_r_spec = pl.BlockSpec((_BR, _D), lambda i: (i, 0))
_p_spec = pl.BlockSpec((_NC, _BR, _D), lambda i: (0, i, 0))
_i_spec = pl.BlockSpec((_BR, 1), lambda i: (i, 0))


def _tc1(p, cnt, x, Wl, bl, Wr):
    return pl.pallas_call(
        _tc1_body,
        grid=(_N // _BR,),
        in_specs=[_p_spec, _p_spec, _r_spec, _w_spec, _b_spec, _w_spec],
        out_specs=[_r_spec, _i_spec],
        out_shape=[jax.ShapeDtypeStruct((_N, _D), jnp.float32),
                   jax.ShapeDtypeStruct((_N, 1), jnp.float32)],
    )(p, cnt, x, Wl, bl, Wr)


def _tc2(p, inv, x, Wl, bl, Wr):
    return pl.pallas_call(
        _tc2_body,
        grid=(_N // _BR,),
        in_specs=[_p_spec, _i_spec, _r_spec, _w_spec, _b_spec, _w_spec],
        out_specs=_r_spec,
        out_shape=jax.ShapeDtypeStruct((_N, _D), jnp.float32),
    )(p, inv, x, Wl, bl, Wr)


def kernel(x, edge_index, Wl1, bl1, Wr1, Wl2, bl2, Wr2):
    npad = _EPAD - _E
    src = jnp.concatenate(
        [edge_index[0], jnp.zeros((npad,), jnp.int32)]).reshape(-1, _CH)
    dst = jnp.concatenate(
        [edge_index[1], jnp.full((npad,), _N, jnp.int32)]).reshape(-1, _CH)

    (cnt,) = _sc_count(x, src, dst)
    (p1,) = _sc_agg(x, src, dst)
    h, inv = _tc1(p1, cnt, x, Wl1, bl1.reshape(1, _D), Wr1)
    (p2,) = _sc_agg(h, src, dst)
    return _tc2(p2, inv, h, Wl2, bl2.reshape(1, _D), Wr2)


# speedup vs baseline: 1.3353x; 1.3353x over previous
import jax
import jax.numpy as jnp
from jax import lax
from jax.experimental import pallas as pl
from jax.experimental.pallas import tpu as pltpu
from jax.experimental.pallas import tpu_sc as plsc

_N = 10000
_E = 320000
_D = 128
_NC = 2
_NS = 16
_NW = _NC * _NS
_CH = 128
_NCHUNK = 80
_EPAD = _NW * _NCHUNK * _CH
_IB = 8
_NBLK = _NCHUNK // _IB
_K0 = 144
_K1 = 2 * _NCHUNK - _K0
_NP = 10240
_RPT = _NP // _NS
_ZC = 128
_L = 16

_mesh = plsc.VectorSubcoreMesh(core_axis_name="c", subcore_axis_name="s")


def _fill_rows(rows_v, value):
    vv = jnp.full((_L,), value, jnp.float32)

    def frow(i, _):
        def fcol(j, _):
            rows_v[i, pl.ds(j * _L, _L)] = vv
            return 0
        return lax.fori_loop(0, _D // _L, fcol, 0)
    lax.fori_loop(0, _ZC, frow, 0)


def _make_sc_agg(count_only):
    def body(table, src, dst, out_p, src_v, dst_v, rows0, rows1, acc,
             sem0, sem1):
        c = lax.axis_index("c")
        s = lax.axis_index("s")
        wid = s * _NC + c
        base = s * _RPT

        _fill_rows(rows0, 0.0)
        for k in range(_RPT // _ZC):
            pltpu.sync_copy(rows0, acc.at[pl.ds(base + k * _ZC, _ZC)])
        if count_only:
            _fill_rows(rows0, 1.0)
        plsc.subcore_barrier()

        rows = (rows0, rows1)
        sems = (sem0, sem1)

        if count_only:
            def block(b, _):
                pltpu.sync_copy(dst.at[pl.ds(wid * _NCHUNK + b * _IB, _IB)],
                                dst_v.at[0])

                def chunk(j, _):
                    pltpu.sync_copy(rows0, acc.at[dst_v.at[0, j]], add=True)
                    return 0
                lax.fori_loop(0, _IB, chunk, 0)
                return 0
            lax.fori_loop(0, _NBLK, block, 0)
        else:
            nblk = jnp.where(c == 0, _K0 // _IB, _K1 // _IB)
            start = jnp.where(c == 0, s * _K0, _NS * _K0 + s * _K1)

            def stage(b):
                ibase = start + b * _IB
                pltpu.sync_copy(src.at[pl.ds(ibase, _IB)], src_v.at[b % 2])
                pltpu.sync_copy(dst.at[pl.ds(ibase, _IB)], dst_v.at[b % 2])

            def issue(b, j, par):
                return pltpu.async_copy(table.at[src_v.at[b % 2, j]],
                                        rows[par], sems[par])

            @pl.when(nblk > 0)
            def _():
                stage(0)
                issue(0, 0, 0)

            def block(b, _):
                @pl.when(b + 1 < nblk)
                def _():
                    stage(b + 1)
                for j in range(_IB):
                    par = j % 2
                    if j + 1 < _IB:
                        issue(b, j + 1, (j + 1) % 2)
                    else:
                        @pl.when(b + 1 < nblk)
                        def _():
                            issue(b + 1, 0, (j + 1) % 2)
                    pltpu.make_async_copy(table.at[src_v.at[b % 2, j]],
                                          rows[par], sems[par]).wait()
                    pltpu.sync_copy(rows[par],
                                    acc.at[dst_v.at[b % 2, j]], add=True)
                return 0
            lax.fori_loop(0, nblk, block, 0)
        plsc.subcore_barrier()

        pltpu.sync_copy(acc.at[pl.ds(base, _RPT)],
                        out_p.at[c, pl.ds(base, _RPT)])

    return pl.kernel(
        body,
        out_type=[jax.ShapeDtypeStruct((_NC, _NP, _D), jnp.float32)],
        mesh=_mesh,
        scratch_types=[
            pltpu.VMEM((2, _IB, _CH), jnp.int32),
            pltpu.VMEM((2, _IB, _CH), jnp.int32),
            pltpu.VMEM((_ZC, _D), jnp.float32),
            pltpu.VMEM((_ZC, _D), jnp.float32),
            pltpu.VMEM_SHARED((_NP, _D), jnp.float32),
            pltpu.SemaphoreType.DMA,
            pltpu.SemaphoreType.DMA,
        ],
    )


_sc_agg = _make_sc_agg(False)
_sc_count = _make_sc_agg(True)

_BR = 1000
_dn = (((1,), (1,)), ((), ()))


def _tc1_body(p_ref, c_ref, x_ref, wl_ref, bl_ref, wr_ref, h_ref, inv_ref):
    ps = p_ref[0] + p_ref[1]
    cn = c_ref[0, :, 0:1] + c_ref[1, :, 0:1]
    inv = 1.0 / jnp.maximum(cn, 1.0)
    agg = ps * inv
    y = lax.dot_general(agg, wl_ref[...], _dn, preferred_element_type=jnp.float32)
    y = y + bl_ref[...] + lax.dot_general(x_ref[...], wr_ref[...], _dn,
                                          preferred_element_type=jnp.float32)
    h_ref[...] = jnp.maximum(y, 0.0)
    inv_ref[...] = inv


def _tc2_body(p_ref, inv_ref, x_ref, wl_ref, bl_ref, wr_ref, o_ref):
    agg = (p_ref[0] + p_ref[1]) * inv_ref[...]
    y = lax.dot_general(agg, wl_ref[...], _dn, preferred_element_type=jnp.float32)
    y = y + bl_ref[...] + lax.dot_general(x_ref[...], wr_ref[...], _dn,
                                          preferred_element_type=jnp.float32)
    o_ref[...] = y


_w_spec = pl.BlockSpec((_D, _D), lambda i: (0, 0))
_b_spec = pl.BlockSpec((1, _D), lambda i: (0, 0))
_r_spec = pl.BlockSpec((_BR, _D), lambda i: (i, 0))
_p_spec = pl.BlockSpec((_NC, _BR, _D), lambda i: (0, i, 0))
_i_spec = pl.BlockSpec((_BR, 1), lambda i: (i, 0))


def _tc1(p, cnt, x, Wl, bl, Wr):
    return pl.pallas_call(
        _tc1_body,
        grid=(_N // _BR,),
        in_specs=[_p_spec, _p_spec, _r_spec, _w_spec, _b_spec, _w_spec],
        out_specs=[_r_spec, _i_spec],
        out_shape=[jax.ShapeDtypeStruct((_N, _D), jnp.float32),
                   jax.ShapeDtypeStruct((_N, 1), jnp.float32)],
    )(p, cnt, x, Wl, bl, Wr)


def _tc2(p, inv, x, Wl, bl, Wr):
    return pl.pallas_call(
        _tc2_body,
        grid=(_N // _BR,),
        in_specs=[_p_spec, _i_spec, _r_spec, _w_spec, _b_spec, _w_spec],
        out_specs=_r_spec,
        out_shape=jax.ShapeDtypeStruct((_N, _D), jnp.float32),
    )(p, inv, x, Wl, bl, Wr)


def kernel(x, edge_index, Wl1, bl1, Wr1, Wl2, bl2, Wr2):
    npad = _EPAD - _E
    src = jnp.concatenate(
        [edge_index[0], jnp.zeros((npad,), jnp.int32)]).reshape(-1, _CH)
    dst = jnp.concatenate(
        [edge_index[1], jnp.full((npad,), _N, jnp.int32)]).reshape(-1, _CH)

    (cnt,) = _sc_count(x, src, dst)
    (p1,) = _sc_agg(x, src, dst)
    h, inv = _tc1(p1, cnt, x, Wl1, bl1.reshape(1, _D), Wr1)
    (p2,) = _sc_agg(h, src, dst)
    return _tc2(p2, inv, h, Wl2, bl2.reshape(1, _D), Wr2)


# speedup vs baseline: 1.3476x; 1.0092x over previous
import jax
import jax.numpy as jnp
from jax import lax
from jax.experimental import pallas as pl
from jax.experimental.pallas import tpu as pltpu
from jax.experimental.pallas import tpu_sc as plsc

_N = 10000
_E = 320000
_D = 128
_NC = 2
_NS = 16
_NW = _NC * _NS
_CH = 128
_NCHUNK = 80
_EPAD = _NW * _NCHUNK * _CH
_IB = 8
_NBLK = _NCHUNK // _IB
_K0 = 152
_K1 = 2 * _NCHUNK - _K0
_NP = 10240
_RPT = _NP // _NS
_ZC = 128
_L = 16

_mesh = plsc.VectorSubcoreMesh(core_axis_name="c", subcore_axis_name="s")


def _fill_rows(rows_v, value):
    vv = jnp.full((_L,), value, jnp.float32)

    def frow(i, _):
        def fcol(j, _):
            rows_v[i, pl.ds(j * _L, _L)] = vv
            return 0
        return lax.fori_loop(0, _D // _L, fcol, 0)
    lax.fori_loop(0, _ZC, frow, 0)


def _make_sc_agg(count_only):
    def body(table, src, dst, out_p, src_v, dst_v, rows0, rows1, acc,
             sem0, sem1):
        c = lax.axis_index("c")
        s = lax.axis_index("s")
        wid = s * _NC + c
        base = s * _RPT

        _fill_rows(rows0, 0.0)
        for k in range(_RPT // _ZC):
            pltpu.sync_copy(rows0, acc.at[pl.ds(base + k * _ZC, _ZC)])
        if count_only:
            _fill_rows(rows0, 1.0)
        plsc.subcore_barrier()

        rows = (rows0, rows1)
        sems = (sem0, sem1)

        if count_only:
            def block(b, _):
                pltpu.sync_copy(dst.at[pl.ds(wid * _NCHUNK + b * _IB, _IB)],
                                dst_v.at[0])

                def chunk(j, _):
                    pltpu.sync_copy(rows0, acc.at[dst_v.at[0, j]], add=True)
                    return 0
                lax.fori_loop(0, _IB, chunk, 0)
                return 0
            lax.fori_loop(0, _NBLK, block, 0)
        else:
            nblk = jnp.where(c == 0, _K0 // _IB, _K1 // _IB)
            start = jnp.where(c == 0, s * _K0, _NS * _K0 + s * _K1)

            def stage(b):
                ibase = start + b * _IB
                pltpu.sync_copy(src.at[pl.ds(ibase, _IB)], src_v.at[b % 2])
                pltpu.sync_copy(dst.at[pl.ds(ibase, _IB)], dst_v.at[b % 2])

            def issue(b, j, par):
                return pltpu.async_copy(table.at[src_v.at[b % 2, j]],
                                        rows[par], sems[par])

            @pl.when(nblk > 0)
            def _():
                stage(0)
                issue(0, 0, 0)

            def block(b, _):
                @pl.when(b + 1 < nblk)
                def _():
                    stage(b + 1)
                for j in range(_IB):
                    par = j % 2
                    if j + 1 < _IB:
                        issue(b, j + 1, (j + 1) % 2)
                    else:
                        @pl.when(b + 1 < nblk)
                        def _():
                            issue(b + 1, 0, (j + 1) % 2)
                    pltpu.make_async_copy(table.at[src_v.at[b % 2, j]],
                                          rows[par], sems[par]).wait()
                    pltpu.sync_copy(rows[par],
                                    acc.at[dst_v.at[b % 2, j]], add=True)
                return 0
            lax.fori_loop(0, nblk, block, 0)
        plsc.subcore_barrier()

        pltpu.sync_copy(acc.at[pl.ds(base, _RPT)],
                        out_p.at[c, pl.ds(base, _RPT)])

    return pl.kernel(
        body,
        out_type=[jax.ShapeDtypeStruct((_NC, _NP, _D), jnp.float32)],
        mesh=_mesh,
        scratch_types=[
            pltpu.VMEM((2, _IB, _CH), jnp.int32),
            pltpu.VMEM((2, _IB, _CH), jnp.int32),
            pltpu.VMEM((_ZC, _D), jnp.float32),
            pltpu.VMEM((_ZC, _D), jnp.float32),
            pltpu.VMEM_SHARED((_NP, _D), jnp.float32),
            pltpu.SemaphoreType.DMA,
            pltpu.SemaphoreType.DMA,
        ],
    )


_sc_agg = _make_sc_agg(False)
_sc_count = _make_sc_agg(True)

_BR = 1000
_dn = (((1,), (1,)), ((), ()))


def _tc1_body(p_ref, c_ref, x_ref, wl_ref, bl_ref, wr_ref, h_ref, inv_ref):
    ps = p_ref[0] + p_ref[1]
    cn = c_ref[0, :, 0:1] + c_ref[1, :, 0:1]
    inv = 1.0 / jnp.maximum(cn, 1.0)
    agg = ps * inv
    y = lax.dot_general(agg, wl_ref[...], _dn, preferred_element_type=jnp.float32)
    y = y + bl_ref[...] + lax.dot_general(x_ref[...], wr_ref[...], _dn,
                                          preferred_element_type=jnp.float32)
    h_ref[...] = jnp.maximum(y, 0.0)
    inv_ref[...] = inv


def _tc2_body(p_ref, inv_ref, x_ref, wl_ref, bl_ref, wr_ref, o_ref):
    agg = (p_ref[0] + p_ref[1]) * inv_ref[...]
    y = lax.dot_general(agg, wl_ref[...], _dn, preferred_element_type=jnp.float32)
    y = y + bl_ref[...] + lax.dot_general(x_ref[...], wr_ref[...], _dn,
                                          preferred_element_type=jnp.float32)
    o_ref[...] = y


_w_spec = pl.BlockSpec((_D, _D), lambda i: (0, 0))
_b_spec = pl.BlockSpec((1, _D), lambda i: (0, 0))
_r_spec = pl.BlockSpec((_BR, _D), lambda i: (i, 0))
_p_spec = pl.BlockSpec((_NC, _BR, _D), lambda i: (0, i, 0))
_i_spec = pl.BlockSpec((_BR, 1), lambda i: (i, 0))


def _tc1(p, cnt, x, Wl, bl, Wr):
    return pl.pallas_call(
        _tc1_body,
        grid=(_N // _BR,),
        in_specs=[_p_spec, _p_spec, _r_spec, _w_spec, _b_spec, _w_spec],
        out_specs=[_r_spec, _i_spec],
        out_shape=[jax.ShapeDtypeStruct((_N, _D), jnp.float32),
                   jax.ShapeDtypeStruct((_N, 1), jnp.float32)],
    )(p, cnt, x, Wl, bl, Wr)


def _tc2(p, inv, x, Wl, bl, Wr):
    return pl.pallas_call(
        _tc2_body,
        grid=(_N // _BR,),
        in_specs=[_p_spec, _i_spec, _r_spec, _w_spec, _b_spec, _w_spec],
        out_specs=_r_spec,
        out_shape=jax.ShapeDtypeStruct((_N, _D), jnp.float32),
    )(p, inv, x, Wl, bl, Wr)


def kernel(x, edge_index, Wl1, bl1, Wr1, Wl2, bl2, Wr2):
    npad = _EPAD - _E
    src = jnp.concatenate(
        [edge_index[0], jnp.zeros((npad,), jnp.int32)]).reshape(-1, _CH)
    dst = jnp.concatenate(
        [edge_index[1], jnp.full((npad,), _N, jnp.int32)]).reshape(-1, _CH)

    (cnt,) = _sc_count(x, src, dst)
    (p1,) = _sc_agg(x, src, dst)
    h, inv = _tc1(p1, cnt, x, Wl1, bl1.reshape(1, _D), Wr1)
    (p2,) = _sc_agg(h, src, dst)
    return _tc2(p2, inv, h, Wl2, bl2.reshape(1, _D), Wr2)
